# baseline (device time: 62930 ns/iter reference)
import jax
import jax.numpy as jnp
from jax import lax
from jax.experimental import pallas as pl
from jax.experimental.pallas import tpu as pltpu

N_DEV = 8

X = (0, 1, 1, 0, 0, 1, 1, 0)
Y = (0, 0, 1, 1, 0, 0, 1, 1)
Z = (0, 0, 0, 0, 1, 1, 1, 1)
COORDS = (X, Y, Z)
MASK = (1, 3, 4)

ORDERS = (
    (0, 1, 2), (0, 1, 2), (1, 2, 0), (2, 0, 1),
    (0, 1, 2), (0, 1, 2), (1, 2, 0), (2, 0, 1),
)


def kernel(x, w_mat):
    m_total, k_local = x.shape
    k2, n = w_mat.shape
    m_per = m_total // N_DEV

    def body(x_ref, w_ref, out_ref, send_ref, recv_ref, send_sems, recv_sems):
        my = lax.axis_index("i")
        px = (my ^ (my >> 1)) & 1
        py = (my >> 1) & 1
        pz = (my >> 2) & 1
        pc = (px, py, pz)

        barrier_sem = pltpu.get_barrier_semaphore()
        for mask in MASK:
            pl.semaphore_signal(
                barrier_sem, inc=1,
                device_id=(my ^ mask,), device_id_type=pl.DeviceIdType.MESH,
            )
        pl.semaphore_wait(barrier_sem, 3)

        w = w_ref[...].astype(jnp.bfloat16)

        def send_rdma(q, step, axis):
            return pltpu.make_async_remote_copy(
                src_ref=send_ref.at[q],
                dst_ref=recv_ref.at[q, step],
                send_sem=send_sems.at[q],
                recv_sem=recv_sems.at[q, step],
                device_id=(my ^ MASK[axis],),
                device_id_type=pl.DeviceIdType.MESH,
            )

        def wait_recv(q, step):
            pltpu.make_async_remote_copy(
                src_ref=send_ref.at[q],
                dst_ref=recv_ref.at[q, step],
                send_sem=send_sems.at[q],
                recv_sem=recv_sems.at[q, step],
                device_id=(my,),
                device_id_type=pl.DeviceIdType.MESH,
            ).wait_recv()

        def add_into_send(q, *recv_steps):
            acc = send_ref[q, ...].astype(jnp.float32)
            for s in recv_steps:
                acc = acc + recv_ref[q, s, ...].astype(jnp.float32)
            send_ref[q, ...] = acc.astype(jnp.bfloat16)

        def cdiff(q):
            d1, d2, d3 = ORDERS[q]
            return (
                pc[d1] != COORDS[d1][q],
                pc[d2] != COORDS[d2][q],
                pc[d3] != COORDS[d3][q],
            )

        for q in range(N_DEV):
            xa = x_ref[q * m_per:(q + 1) * m_per, :].astype(jnp.bfloat16)
            send_ref[q, ...] = jnp.dot(
                xa, w, preferred_element_type=jnp.float32
            ).astype(jnp.bfloat16)

        for pred_fn in (
            lambda c1, c2, c3: jnp.logical_and(c1, c2),
            lambda c1, c2, c3: jnp.logical_and(
                c1, jnp.logical_and(~c2, c3)),
            lambda c1, c2, c3: jnp.logical_and(
                c1, jnp.logical_and(~c2, ~c3)),
        ):
            for q in range(N_DEV):
                c1, c2, c3 = cdiff(q)

                @pl.when(pred_fn(c1, c2, c3))
                def _(q=q):
                    send_rdma(q, 0, ORDERS[q][0]).start()

        for pred_fn in (
            lambda c1, c2, c3: jnp.logical_and(
                ~c1, jnp.logical_and(c2, c3)),
            lambda c1, c2, c3: jnp.logical_and(
                ~c1, jnp.logical_and(c2, ~c3)),
        ):
            for q in range(N_DEV):
                c1, c2, c3 = cdiff(q)

                @pl.when(pred_fn(c1, c2, c3))
                def _(q=q):
                    wait_recv(q, 0)
                    add_into_send(q, 0)
                    send_rdma(q, 1, ORDERS[q][1]).start()

        for q in range(N_DEV):
            c1, c2, c3 = cdiff(q)

            @pl.when(jnp.logical_and(~c1, jnp.logical_and(~c2, c3)))
            def _(q=q):
                wait_recv(q, 0)
                wait_recv(q, 1)
                add_into_send(q, 0, 1)
                send_rdma(q, 2, ORDERS[q][2]).start()

        for q in range(N_DEV):
            @pl.when(my == q)
            def _(q=q):
                wait_recv(q, 0)
                wait_recv(q, 1)
                wait_recv(q, 2)
                out_ref[...] = (
                    send_ref[q, ...].astype(jnp.float32)
                    + recv_ref[q, 0, ...].astype(jnp.float32)
                    + recv_ref[q, 1, ...].astype(jnp.float32)
                    + recv_ref[q, 2, ...].astype(jnp.float32)
                )

        for q in range(N_DEV):
            @pl.when(my != q)
            def _(q=q):
                pltpu.make_async_remote_copy(
                    src_ref=send_ref.at[q],
                    dst_ref=recv_ref.at[q, 0],
                    send_sem=send_sems.at[q],
                    recv_sem=recv_sems.at[q, 0],
                    device_id=(my,),
                    device_id_type=pl.DeviceIdType.MESH,
                ).wait_send()

    return pl.pallas_call(
        body,
        out_shape=jax.ShapeDtypeStruct((m_per, n), jnp.float32),
        in_specs=[
            pl.BlockSpec(memory_space=pltpu.VMEM),
            pl.BlockSpec(memory_space=pltpu.VMEM),
        ],
        out_specs=pl.BlockSpec(memory_space=pltpu.VMEM),
        scratch_shapes=[
            pltpu.VMEM((N_DEV, m_per, n), jnp.bfloat16),
            pltpu.VMEM((N_DEV, 3, m_per, n), jnp.bfloat16),
            pltpu.SemaphoreType.DMA((N_DEV,)),
            pltpu.SemaphoreType.DMA((N_DEV, 3)),
        ],
        compiler_params=pltpu.CompilerParams(
            collective_id=0, vmem_limit_bytes=100 * 1024 * 1024
        ),
    )(x, w_mat)


# device time: 50372 ns/iter; 1.2493x vs baseline; 1.2493x over previous
import jax
import jax.numpy as jnp
from jax import lax
from jax.experimental import pallas as pl
from jax.experimental.pallas import tpu as pltpu

N_DEV = 8

X = (0, 1, 1, 0, 0, 1, 1, 0)
Y = (0, 0, 1, 1, 0, 0, 1, 1)
Z = (0, 0, 0, 0, 1, 1, 1, 1)
COORDS = (X, Y, Z)
MASK = (1, 3, 4)

ORDERS2 = (
    ((0, 1, 2), (2, 1, 0)),
    ((2, 0, 1), (1, 0, 2)),
    ((2, 0, 1), (1, 0, 2)),
    ((0, 1, 2), (1, 0, 2)),
    ((1, 2, 0), (1, 0, 2)),
    ((2, 0, 1), (0, 2, 1)),
    ((2, 0, 1), (0, 2, 1)),
    ((1, 0, 2), (0, 2, 1)),
)


def kernel(x, w_mat):
    m_total, k_local = x.shape
    k2, n = w_mat.shape
    m_per = m_total // N_DEV
    nh = n // 2

    def body(x_ref, w_ref, out_ref, send_ref, recv_ref, send_sems, recv_sems):
        my = lax.axis_index("i")
        px = (my ^ (my >> 1)) & 1
        py = (my >> 1) & 1
        pz = (my >> 2) & 1
        pc = (px, py, pz)

        barrier_sem = pltpu.get_barrier_semaphore()
        for mask in MASK:
            pl.semaphore_signal(
                barrier_sem, inc=1,
                device_id=(my ^ mask,), device_id_type=pl.DeviceIdType.MESH,
            )
        pl.semaphore_wait(barrier_sem, 3)

        w = w_ref[...].astype(jnp.bfloat16)

        def sslot(q, h):
            return send_ref.at[q, :, h * nh:(h + 1) * nh]

        def rslot(q, s, h):
            return recv_ref.at[q, s, :, h * nh:(h + 1) * nh]

        def descriptor(q, s, h, dev):
            return pltpu.make_async_remote_copy(
                src_ref=sslot(q, h),
                dst_ref=rslot(q, s, h),
                send_sem=send_sems.at[q, h],
                recv_sem=recv_sems.at[q, s, h],
                device_id=(dev,),
                device_id_type=pl.DeviceIdType.MESH,
            )

        def wait_recv(q, s, h):
            descriptor(q, s, h, my).wait_recv()

        def combine(q, h, *steps):
            acc = send_ref[q, :, h * nh:(h + 1) * nh].astype(jnp.float32)
            for s in steps:
                acc = acc + recv_ref[q, s, :, h * nh:(h + 1) * nh].astype(
                    jnp.float32
                )
            send_ref[q, :, h * nh:(h + 1) * nh] = acc.astype(jnp.bfloat16)

        def cdiff(q, h):
            d1, d2, d3 = ORDERS2[q][h]
            return (
                pc[d1] != COORDS[d1][q],
                pc[d2] != COORDS[d2][q],
                pc[d3] != COORDS[d3][q],
            )

        for q in range(N_DEV):
            xa = x_ref[q * m_per:(q + 1) * m_per, :].astype(jnp.bfloat16)
            send_ref[q, ...] = jnp.dot(
                xa, w, preferred_element_type=jnp.float32
            ).astype(jnp.bfloat16)

        for q in range(N_DEV):
            for h in range(2):
                c1, _, _ = cdiff(q, h)

                @pl.when(c1)
                def _(q=q, h=h):
                    descriptor(q, 0, h, my ^ MASK[ORDERS2[q][h][0]]).start()

        for q in range(N_DEV):
            for h in range(2):
                c1, c2, _ = cdiff(q, h)

                @pl.when(jnp.logical_and(~c1, c2))
                def _(q=q, h=h):
                    wait_recv(q, 0, h)
                    combine(q, h, 0)
                    descriptor(q, 1, h, my ^ MASK[ORDERS2[q][h][1]]).start()

        for q in range(N_DEV):
            for h in range(2):
                c1, c2, c3 = cdiff(q, h)

                @pl.when(jnp.logical_and(~c1, jnp.logical_and(~c2, c3)))
                def _(q=q, h=h):
                    wait_recv(q, 0, h)
                    wait_recv(q, 1, h)
                    combine(q, h, 0, 1)
                    descriptor(q, 2, h, my ^ MASK[ORDERS2[q][h][2]]).start()

        for q in range(N_DEV):
            @pl.when(my == q)
            def _(q=q):
                for h in range(2):
                    wait_recv(q, 0, h)
                    wait_recv(q, 1, h)
                    wait_recv(q, 2, h)
                    sl = slice(h * nh, (h + 1) * nh)
                    out_ref[:, sl] = (
                        send_ref[q, :, sl].astype(jnp.float32)
                        + recv_ref[q, 0, :, sl].astype(jnp.float32)
                        + recv_ref[q, 1, :, sl].astype(jnp.float32)
                        + recv_ref[q, 2, :, sl].astype(jnp.float32)
                    )

        for q in range(N_DEV):
            @pl.when(my != q)
            def _(q=q):
                for h in range(2):
                    descriptor(q, 0, h, my).wait_send()

    return pl.pallas_call(
        body,
        out_shape=jax.ShapeDtypeStruct((m_per, n), jnp.float32),
        in_specs=[
            pl.BlockSpec(memory_space=pltpu.VMEM),
            pl.BlockSpec(memory_space=pltpu.VMEM),
        ],
        out_specs=pl.BlockSpec(memory_space=pltpu.VMEM),
        scratch_shapes=[
            pltpu.VMEM((N_DEV, m_per, n), jnp.bfloat16),
            pltpu.VMEM((N_DEV, 3, m_per, n), jnp.bfloat16),
            pltpu.SemaphoreType.DMA((N_DEV, 2)),
            pltpu.SemaphoreType.DMA((N_DEV, 3, 2)),
        ],
        compiler_params=pltpu.CompilerParams(
            collective_id=0, vmem_limit_bytes=100 * 1024 * 1024
        ),
    )(x, w_mat)


# device time: 47270 ns/iter; 1.3313x vs baseline; 1.0656x over previous
import jax
import jax.numpy as jnp
from jax import lax
from jax.experimental import pallas as pl
from jax.experimental.pallas import tpu as pltpu

N_DEV = 8

X = (0, 1, 1, 0, 0, 1, 1, 0)
Y = (0, 0, 1, 1, 0, 0, 1, 1)
Z = (0, 0, 0, 0, 1, 1, 1, 1)
COORDS = (X, Y, Z)
MASK = (1, 3, 4)

ORDERS2 = (
    ((2, 0, 1), (2, 1, 0), (2, 0, 1), (1, 0, 2)),
    ((0, 2, 1), (0, 1, 2), (1, 2, 0), (1, 0, 2)),
    ((1, 2, 0), (1, 0, 2), (2, 0, 1), (0, 2, 1)),
    ((2, 0, 1), (0, 2, 1), (1, 0, 2), (0, 2, 1)),
    ((0, 1, 2), (1, 2, 0), (1, 0, 2), (2, 0, 1)),
    ((2, 1, 0), (2, 0, 1), (0, 2, 1), (1, 0, 2)),
    ((0, 1, 2), (2, 1, 0), (0, 1, 2), (2, 1, 0)),
    ((1, 0, 2), (1, 2, 0), (2, 0, 1), (0, 1, 2)),
)


def kernel(x, w_mat):
    m_total, k_local = x.shape
    k2, n = w_mat.shape
    m_per = m_total // N_DEV
    nh = n // 4

    def body(x_ref, w_ref, out_ref, send_ref, recv_ref, send_sems, recv_sems):
        my = lax.axis_index("i")
        px = (my ^ (my >> 1)) & 1
        py = (my >> 1) & 1
        pz = (my >> 2) & 1
        pc = (px, py, pz)

        barrier_sem = pltpu.get_barrier_semaphore()
        for mask in MASK:
            pl.semaphore_signal(
                barrier_sem, inc=1,
                device_id=(my ^ mask,), device_id_type=pl.DeviceIdType.MESH,
            )
        pl.semaphore_wait(barrier_sem, 3)

        w = w_ref[...].astype(jnp.bfloat16)

        def sslot(q, h):
            return send_ref.at[q, :, h * nh:(h + 1) * nh]

        def rslot(q, s, h):
            return recv_ref.at[q, s, :, h * nh:(h + 1) * nh]

        def descriptor(q, s, h, dev):
            return pltpu.make_async_remote_copy(
                src_ref=sslot(q, h),
                dst_ref=rslot(q, s, h),
                send_sem=send_sems.at[q, h],
                recv_sem=recv_sems.at[q, s, h],
                device_id=(dev,),
                device_id_type=pl.DeviceIdType.MESH,
            )

        def wait_recv(q, s, h):
            descriptor(q, s, h, my).wait_recv()

        def combine(q, h, *steps):
            acc = send_ref[q, :, h * nh:(h + 1) * nh].astype(jnp.float32)
            for s in steps:
                acc = acc + recv_ref[q, s, :, h * nh:(h + 1) * nh].astype(
                    jnp.float32
                )
            send_ref[q, :, h * nh:(h + 1) * nh] = acc.astype(jnp.bfloat16)

        def cdiff(q, h):
            d1, d2, d3 = ORDERS2[q][h]
            return (
                pc[d1] != COORDS[d1][q],
                pc[d2] != COORDS[d2][q],
                pc[d3] != COORDS[d3][q],
            )

        for q in range(N_DEV):
            xa = x_ref[q * m_per:(q + 1) * m_per, :].astype(jnp.bfloat16)
            send_ref[q, ...] = jnp.dot(
                xa, w, preferred_element_type=jnp.float32
            ).astype(jnp.bfloat16)

        for q in range(N_DEV):
            for h in range(4):
                c1, _, _ = cdiff(q, h)

                @pl.when(c1)
                def _(q=q, h=h):
                    descriptor(q, 0, h, my ^ MASK[ORDERS2[q][h][0]]).start()

        for q in range(N_DEV):
            for h in range(4):
                c1, c2, _ = cdiff(q, h)

                @pl.when(jnp.logical_and(~c1, c2))
                def _(q=q, h=h):
                    wait_recv(q, 0, h)
                    combine(q, h, 0)
                    descriptor(q, 1, h, my ^ MASK[ORDERS2[q][h][1]]).start()

        for q in range(N_DEV):
            for h in range(4):
                c1, c2, c3 = cdiff(q, h)

                @pl.when(jnp.logical_and(~c1, jnp.logical_and(~c2, c3)))
                def _(q=q, h=h):
                    wait_recv(q, 0, h)
                    wait_recv(q, 1, h)
                    combine(q, h, 0, 1)
                    descriptor(q, 2, h, my ^ MASK[ORDERS2[q][h][2]]).start()

        for q in range(N_DEV):
            @pl.when(my == q)
            def _(q=q):
                for h in range(4):
                    wait_recv(q, 0, h)
                    wait_recv(q, 1, h)
                    wait_recv(q, 2, h)
                    sl = slice(h * nh, (h + 1) * nh)
                    out_ref[:, sl] = (
                        send_ref[q, :, sl].astype(jnp.float32)
                        + recv_ref[q, 0, :, sl].astype(jnp.float32)
                        + recv_ref[q, 1, :, sl].astype(jnp.float32)
                        + recv_ref[q, 2, :, sl].astype(jnp.float32)
                    )

        for q in range(N_DEV):
            @pl.when(my != q)
            def _(q=q):
                for h in range(4):
                    descriptor(q, 0, h, my).wait_send()

    return pl.pallas_call(
        body,
        out_shape=jax.ShapeDtypeStruct((m_per, n), jnp.float32),
        in_specs=[
            pl.BlockSpec(memory_space=pltpu.VMEM),
            pl.BlockSpec(memory_space=pltpu.VMEM),
        ],
        out_specs=pl.BlockSpec(memory_space=pltpu.VMEM),
        scratch_shapes=[
            pltpu.VMEM((N_DEV, m_per, n), jnp.bfloat16),
            pltpu.VMEM((N_DEV, 3, m_per, n), jnp.bfloat16),
            pltpu.SemaphoreType.DMA((N_DEV, 4)),
            pltpu.SemaphoreType.DMA((N_DEV, 3, 4)),
        ],
        compiler_params=pltpu.CompilerParams(
            collective_id=0, vmem_limit_bytes=100 * 1024 * 1024
        ),
    )(x, w_mat)


# device time: 45232 ns/iter; 1.3913x vs baseline; 1.0451x over previous
import jax
import jax.numpy as jnp
from jax import lax
from jax.experimental import pallas as pl
from jax.experimental.pallas import tpu as pltpu

N_DEV = 8

X = (0, 1, 1, 0, 0, 1, 1, 0)
Y = (0, 0, 1, 1, 0, 0, 1, 1)
Z = (0, 0, 0, 0, 1, 1, 1, 1)
COORDS = (X, Y, Z)
MASK = (1, 3, 4)

ORDERS2 = (
    ((2, 0, 1), (2, 1, 0), (2, 0, 1), (1, 0, 2)),
    ((0, 2, 1), (0, 1, 2), (1, 2, 0), (1, 0, 2)),
    ((1, 2, 0), (1, 0, 2), (2, 0, 1), (0, 2, 1)),
    ((2, 0, 1), (0, 2, 1), (1, 0, 2), (0, 2, 1)),
    ((0, 1, 2), (1, 2, 0), (1, 0, 2), (2, 0, 1)),
    ((2, 1, 0), (2, 0, 1), (0, 2, 1), (1, 0, 2)),
    ((0, 1, 2), (2, 1, 0), (0, 1, 2), (2, 1, 0)),
    ((1, 0, 2), (1, 2, 0), (2, 0, 1), (0, 1, 2)),
)


def kernel(x, w_mat):
    m_total, k_local = x.shape
    k2, n = w_mat.shape
    m_per = m_total // N_DEV
    nh = n // 4

    def body(x_ref, w_ref, out_ref, send_ref, recv_ref, send_sems, recv_sems):
        my = lax.axis_index("i")
        px = (my ^ (my >> 1)) & 1
        py = (my >> 1) & 1
        pz = (my >> 2) & 1
        pc = (px, py, pz)

        barrier_sem = pltpu.get_barrier_semaphore()
        for mask in MASK:
            pl.semaphore_signal(
                barrier_sem, inc=1,
                device_id=(my ^ mask,), device_id_type=pl.DeviceIdType.MESH,
            )
        pl.semaphore_wait(barrier_sem, 3)

        w = w_ref[...].astype(jnp.bfloat16)

        def sslot(q, h):
            return send_ref.at[q, :, h * nh:(h + 1) * nh]

        def rslot(q, s, h):
            return recv_ref.at[q, s, :, h * nh:(h + 1) * nh]

        def descriptor(q, s, h, dev):
            return pltpu.make_async_remote_copy(
                src_ref=sslot(q, h),
                dst_ref=rslot(q, s, h),
                send_sem=send_sems.at[q, h],
                recv_sem=recv_sems.at[q, s, h],
                device_id=(dev,),
                device_id_type=pl.DeviceIdType.MESH,
            )

        def wait_recv(q, s, h):
            descriptor(q, s, h, my).wait_recv()

        def combine(q, h, *steps):
            acc = send_ref[q, :, h * nh:(h + 1) * nh].astype(jnp.float32)
            for s in steps:
                acc = acc + recv_ref[q, s, :, h * nh:(h + 1) * nh].astype(
                    jnp.float32
                )
            send_ref[q, :, h * nh:(h + 1) * nh] = acc.astype(jnp.bfloat16)

        def cdiff(q, h):
            d1, d2, d3 = ORDERS2[q][h]
            return (
                pc[d1] != COORDS[d1][q],
                pc[d2] != COORDS[d2][q],
                pc[d3] != COORDS[d3][q],
            )

        def rr_by_axis(axis_of):
            groups = {0: [], 1: [], 2: []}
            for q in range(N_DEV):
                for h in range(4):
                    groups[axis_of(q, h)].append((q, h))
            out = []
            for i in range(max(len(g) for g in groups.values())):
                for a in range(3):
                    if i < len(groups[a]):
                        out.append(groups[a][i])
            return out

        for q in range(N_DEV):
            xa = x_ref[q * m_per:(q + 1) * m_per, :].astype(jnp.bfloat16)
            send_ref[q, ...] = jnp.dot(
                xa, w, preferred_element_type=jnp.float32
            ).astype(jnp.bfloat16)
            for h in range(4):
                c1, _, _ = cdiff(q, h)

                @pl.when(c1)
                def _(q=q, h=h):
                    descriptor(q, 0, h, my ^ MASK[ORDERS2[q][h][0]]).start()

        for q, h in rr_by_axis(lambda q, h: ORDERS2[q][h][0]):
            c1, c2, _ = cdiff(q, h)

            @pl.when(jnp.logical_and(~c1, c2))
            def _(q=q, h=h):
                wait_recv(q, 0, h)
                combine(q, h, 0)
                descriptor(q, 1, h, my ^ MASK[ORDERS2[q][h][1]]).start()

        for q, h in rr_by_axis(lambda q, h: ORDERS2[q][h][1]):
            c1, c2, c3 = cdiff(q, h)

            @pl.when(jnp.logical_and(~c1, jnp.logical_and(~c2, c3)))
            def _(q=q, h=h):
                wait_recv(q, 0, h)
                wait_recv(q, 1, h)
                combine(q, h, 0, 1)
                descriptor(q, 2, h, my ^ MASK[ORDERS2[q][h][2]]).start()

        for q in range(N_DEV):
            @pl.when(my == q)
            def _(q=q):
                for h in range(4):
                    wait_recv(q, 0, h)
                    wait_recv(q, 1, h)
                    wait_recv(q, 2, h)
                    sl = slice(h * nh, (h + 1) * nh)
                    out_ref[:, sl] = (
                        send_ref[q, :, sl].astype(jnp.float32)
                        + recv_ref[q, 0, :, sl].astype(jnp.float32)
                        + recv_ref[q, 1, :, sl].astype(jnp.float32)
                        + recv_ref[q, 2, :, sl].astype(jnp.float32)
                    )

        for q in range(N_DEV):
            @pl.when(my != q)
            def _(q=q):
                for h in range(4):
                    descriptor(q, 0, h, my).wait_send()

    return pl.pallas_call(
        body,
        out_shape=jax.ShapeDtypeStruct((m_per, n), jnp.float32),
        in_specs=[
            pl.BlockSpec(memory_space=pltpu.VMEM),
            pl.BlockSpec(memory_space=pltpu.VMEM),
        ],
        out_specs=pl.BlockSpec(memory_space=pltpu.VMEM),
        scratch_shapes=[
            pltpu.VMEM((N_DEV, m_per, n), jnp.bfloat16),
            pltpu.VMEM((N_DEV, 3, m_per, n), jnp.bfloat16),
            pltpu.SemaphoreType.DMA((N_DEV, 4)),
            pltpu.SemaphoreType.DMA((N_DEV, 3, 4)),
        ],
        compiler_params=pltpu.CompilerParams(
            collective_id=0, vmem_limit_bytes=100 * 1024 * 1024
        ),
    )(x, w_mat)


# device time: 43918 ns/iter; 1.4329x vs baseline; 1.0299x over previous
import jax
import jax.numpy as jnp
from jax import lax
from jax.experimental import pallas as pl
from jax.experimental.pallas import tpu as pltpu

N_DEV = 8

X = (0, 1, 1, 0, 0, 1, 1, 0)
Y = (0, 0, 1, 1, 0, 0, 1, 1)
Z = (0, 0, 0, 0, 1, 1, 1, 1)
COORDS = (X, Y, Z)
MASK = (1, 3, 4)

ORDERS2 = (
    ((2, 0, 1), (2, 1, 0), (2, 0, 1), (1, 0, 2)),
    ((0, 2, 1), (0, 1, 2), (1, 2, 0), (1, 0, 2)),
    ((1, 2, 0), (1, 0, 2), (2, 0, 1), (0, 2, 1)),
    ((2, 0, 1), (0, 2, 1), (1, 0, 2), (0, 2, 1)),
    ((0, 1, 2), (1, 2, 0), (1, 0, 2), (2, 0, 1)),
    ((2, 1, 0), (2, 0, 1), (0, 2, 1), (1, 0, 2)),
    ((0, 1, 2), (2, 1, 0), (0, 1, 2), (2, 1, 0)),
    ((1, 0, 2), (1, 2, 0), (2, 0, 1), (0, 1, 2)),
)


def kernel(x, w_mat):
    m_total, k_local = x.shape
    k2, n = w_mat.shape
    m_per = m_total // N_DEV
    nh = n // 4

    def body(x_ref, w_ref, out_ref, send_ref, recv_ref, send_sems, recv_sems):
        my = lax.axis_index("i")
        px = (my ^ (my >> 1)) & 1
        py = (my >> 1) & 1
        pz = (my >> 2) & 1
        pc = (px, py, pz)

        barrier_sem = pltpu.get_barrier_semaphore()
        for mask in MASK:
            pl.semaphore_signal(
                barrier_sem, inc=1,
                device_id=(my ^ mask,), device_id_type=pl.DeviceIdType.MESH,
            )

        w = w_ref[...].astype(jnp.bfloat16)

        def sslot(q, h):
            return send_ref.at[q, :, h * nh:(h + 1) * nh]

        def rslot(q, s, h):
            return recv_ref.at[q, s, :, h * nh:(h + 1) * nh]

        def descriptor(q, s, h, dev):
            return pltpu.make_async_remote_copy(
                src_ref=sslot(q, h),
                dst_ref=rslot(q, s, h),
                send_sem=send_sems.at[q, h],
                recv_sem=recv_sems.at[q, s, h],
                device_id=(dev,),
                device_id_type=pl.DeviceIdType.MESH,
            )

        def wait_recv(q, s, h):
            descriptor(q, s, h, my).wait_recv()

        def combine(q, h, *steps):
            acc = send_ref[q, :, h * nh:(h + 1) * nh].astype(jnp.float32)
            for s in steps:
                acc = acc + recv_ref[q, s, :, h * nh:(h + 1) * nh].astype(
                    jnp.float32
                )
            send_ref[q, :, h * nh:(h + 1) * nh] = acc.astype(jnp.bfloat16)

        def cdiff(q, h):
            d1, d2, d3 = ORDERS2[q][h]
            return (
                pc[d1] != COORDS[d1][q],
                pc[d2] != COORDS[d2][q],
                pc[d3] != COORDS[d3][q],
            )

        def rr_by_axis(axis_of):
            groups = {0: [], 1: [], 2: []}
            for q in range(N_DEV):
                for h in range(4):
                    groups[axis_of(q, h)].append((q, h))
            out = []
            for i in range(max(len(g) for g in groups.values())):
                for a in range(3):
                    if i < len(groups[a]):
                        out.append(groups[a][i])
            return out

        for q in range(N_DEV):
            xa = x_ref[q * m_per:(q + 1) * m_per, :].astype(jnp.bfloat16)
            send_ref[q, ...] = jnp.dot(
                xa, w, preferred_element_type=jnp.float32
            ).astype(jnp.bfloat16)
            if q == 0:
                pl.semaphore_wait(barrier_sem, 3)
            for h in range(4):
                c1, _, _ = cdiff(q, h)

                @pl.when(c1)
                def _(q=q, h=h):
                    descriptor(q, 0, h, my ^ MASK[ORDERS2[q][h][0]]).start()

        for q, h in rr_by_axis(lambda q, h: ORDERS2[q][h][0]):
            c1, c2, _ = cdiff(q, h)

            @pl.when(jnp.logical_and(~c1, c2))
            def _(q=q, h=h):
                wait_recv(q, 0, h)
                combine(q, h, 0)
                descriptor(q, 1, h, my ^ MASK[ORDERS2[q][h][1]]).start()

        for q, h in rr_by_axis(lambda q, h: ORDERS2[q][h][1]):
            c1, c2, c3 = cdiff(q, h)

            @pl.when(jnp.logical_and(~c1, jnp.logical_and(~c2, c3)))
            def _(q=q, h=h):
                wait_recv(q, 0, h)
                wait_recv(q, 1, h)
                combine(q, h, 0, 1)
                descriptor(q, 2, h, my ^ MASK[ORDERS2[q][h][2]]).start()

        for q in range(N_DEV):
            @pl.when(my == q)
            def _(q=q):
                for h in range(4):
                    wait_recv(q, 0, h)
                    combine(q, h, 0)
                for h in range(4):
                    wait_recv(q, 1, h)
                    combine(q, h, 1)
                for h in range(4):
                    wait_recv(q, 2, h)
                    sl = slice(h * nh, (h + 1) * nh)
                    out_ref[:, sl] = (
                        send_ref[q, :, sl].astype(jnp.float32)
                        + recv_ref[q, 2, :, sl].astype(jnp.float32)
                    )

        for q in range(N_DEV):
            @pl.when(my != q)
            def _(q=q):
                for h in range(4):
                    descriptor(q, 0, h, my).wait_send()

    return pl.pallas_call(
        body,
        out_shape=jax.ShapeDtypeStruct((m_per, n), jnp.float32),
        in_specs=[
            pl.BlockSpec(memory_space=pltpu.VMEM),
            pl.BlockSpec(memory_space=pltpu.VMEM),
        ],
        out_specs=pl.BlockSpec(memory_space=pltpu.VMEM),
        scratch_shapes=[
            pltpu.VMEM((N_DEV, m_per, n), jnp.bfloat16),
            pltpu.VMEM((N_DEV, 3, m_per, n), jnp.bfloat16),
            pltpu.SemaphoreType.DMA((N_DEV, 4)),
            pltpu.SemaphoreType.DMA((N_DEV, 3, 4)),
        ],
        compiler_params=pltpu.CompilerParams(
            collective_id=0, vmem_limit_bytes=100 * 1024 * 1024
        ),
    )(x, w_mat)
